# baseline (speedup 1.0000x reference)
import functools
import math

import jax
import jax.numpy as jnp
from jax import lax
from jax.experimental.layout import Format, Layout
from jax.experimental import pallas as pl
from jax.experimental.pallas import tpu as pltpu
from jax.experimental.pallas import tpu_sc as plsc

VOCAB_SIZE = 100000
D_MODEL = 128
BATCH = 4096
SEQ_LEN = 50
SCALE = math.sqrt(D_MODEL)

NC = 2
NS = 16
NW = NC * NS

TOTAL = BATCH * SEQ_LEN
PER_W = TOTAL // NW
G_CHUNK = 100
BLOCK = 2 * G_CHUNK
N_BLOCKS = PER_W // BLOCK
N_IDX = PER_W // G_CHUNK
NBUF = 4


def _body(x_hbm, w_hbm, out_hbm, idx_v, rows_v, gsem, ssem):
    wid = lax.axis_index("s") * NC + lax.axis_index("c")
    base = wid * PER_W
    pltpu.sync_copy(x_hbm.at[wid], idx_v)

    def gather(s, buf, wait):
        for i in range(2):
            src = w_hbm.at[idx_v.at[2 * s + i]]
            dst = rows_v.at[buf, pl.ds(i * G_CHUNK, G_CHUNK)]
            if wait:
                pltpu.make_async_copy(src, dst, gsem).wait()
            else:
                pltpu.async_copy(src, dst, gsem)

    def scatter(s, buf, wait):
        src = rows_v.at[buf]
        dst = out_hbm.at[pl.ds(base + s * BLOCK, BLOCK)]
        if wait:
            pltpu.make_async_copy(src, dst, ssem).wait()
        else:
            pltpu.async_copy(src, dst, ssem)

    for b in range(NBUF - 1):
        gather(b, b, wait=False)

    @pl.loop(0, N_BLOCKS, step=NBUF)
    def outer(s0):
        for k in range(NBUF):
            s = s0 + k
            prev = (k - 1) % NBUF
            gather(s, k, wait=True)

            @pl.when(s > 0)
            def _():
                scatter(s - 1, prev, wait=True)

            @pl.when(s + NBUF - 1 < N_BLOCKS)
            def _():
                gather(s + NBUF - 1, prev, wait=False)

            @plsc.parallel_loop(0, BLOCK, unroll=4)
            def scale_row(r):
                for j in range(D_MODEL // 16):
                    rows_v[k, r, pl.ds(j * 16, 16)] = (
                        rows_v[k, r, pl.ds(j * 16, 16)] * SCALE
                    )

            scatter(s, k, wait=False)

    scatter(N_BLOCKS - 1, (N_BLOCKS - 1) % NBUF, wait=True)


def _impl(x, weight):
    xf = x.reshape(NW, N_IDX, G_CHUNK)
    mesh = plsc.VectorSubcoreMesh(
        core_axis_name="c", subcore_axis_name="s", num_cores=NC, num_subcores=NS
    )
    out = pl.kernel(
        _body,
        out_type=jax.ShapeDtypeStruct((TOTAL, D_MODEL), jnp.float32),
        mesh=mesh,
        scratch_types=[
            pltpu.VMEM((N_IDX, G_CHUNK), jnp.int32),
            pltpu.VMEM((NBUF, BLOCK, D_MODEL), jnp.float32),
            pltpu.SemaphoreType.DMA,
            pltpu.SemaphoreType.DMA,
        ],
    )(xf, weight)
    return out.reshape(BATCH, SEQ_LEN, D_MODEL)


@functools.cache
def _jitted():
    fmt = Format(
        Layout(major_to_minor=(0, 1, 2), tiling=()),
        jax.sharding.SingleDeviceSharding(jax.devices()[0]),
    )
    return jax.jit(_impl, out_shardings=fmt)


def kernel(x, weight):
    return _jitted()(x, weight)


# speedup vs baseline: 1.7452x; 1.7452x over previous
import math

import jax
import jax.numpy as jnp
from jax import lax
from jax.experimental import pallas as pl
from jax.experimental.pallas import tpu as pltpu
from jax.experimental.pallas import tpu_sc as plsc

VOCAB_SIZE = 100000
D_MODEL = 128
BATCH = 4096
SEQ_LEN = 50
SCALE = math.sqrt(D_MODEL)

NC = 2
NS = 16
NW = NC * NS

B_PER_W = BATCH // NW
NBUF = 4


def _body(x_hbm, w_hbm, out_hbm, idx_v, rows_v, gsem, ssem):
    wid = lax.axis_index("s") * NC + lax.axis_index("c")
    b_base = wid * B_PER_W
    pltpu.sync_copy(x_hbm.at[wid], idx_v)

    def gather(b, buf, wait):
        src = w_hbm.at[idx_v.at[b]]
        dst = rows_v.at[buf]
        if wait:
            pltpu.make_async_copy(src, dst, gsem).wait()
        else:
            pltpu.async_copy(src, dst, gsem)

    def scatter(b, buf, wait):
        src = rows_v.at[buf]
        dst = out_hbm.at[b_base + b]
        if wait:
            pltpu.make_async_copy(src, dst, ssem).wait()
        else:
            pltpu.async_copy(src, dst, ssem)

    for b in range(NBUF - 1):
        gather(b, b, wait=False)

    @pl.loop(0, B_PER_W, step=NBUF)
    def outer(b0):
        for k in range(NBUF):
            b = b0 + k
            prev = (k - 1) % NBUF
            gather(b, k, wait=True)

            @pl.when(b > 0)
            def _():
                scatter(b - 1, prev, wait=True)

            @pl.when(b + NBUF - 1 < B_PER_W)
            def _():
                gather(b + NBUF - 1, prev, wait=False)

            @plsc.parallel_loop(0, SEQ_LEN, unroll=2)
            def scale_row(r):
                for j in range(D_MODEL // 16):
                    rows_v[k, r, pl.ds(j * 16, 16)] = (
                        rows_v[k, r, pl.ds(j * 16, 16)] * SCALE
                    )

            scatter(b, k, wait=False)

    scatter(B_PER_W - 1, (B_PER_W - 1) % NBUF, wait=True)


@jax.jit
def kernel(x, weight):
    xf = x.reshape(NW, B_PER_W, SEQ_LEN)
    mesh = plsc.VectorSubcoreMesh(
        core_axis_name="c", subcore_axis_name="s", num_cores=NC, num_subcores=NS
    )
    return pl.kernel(
        _body,
        out_type=jax.ShapeDtypeStruct((BATCH, SEQ_LEN, D_MODEL), jnp.float32),
        mesh=mesh,
        compiler_params=pltpu.CompilerParams(use_tc_tiling_on_sc=True),
        scratch_types=[
            pltpu.VMEM((B_PER_W, SEQ_LEN), jnp.int32),
            pltpu.VMEM((NBUF, SEQ_LEN, D_MODEL), jnp.float32),
            pltpu.SemaphoreType.DMA,
            pltpu.SemaphoreType.DMA,
        ],
    )(xf, weight)


# speedup vs baseline: 3.0932x; 1.7725x over previous
import math

import jax
import jax.numpy as jnp
from jax import lax
from jax.experimental import pallas as pl
from jax.experimental.pallas import tpu as pltpu
from jax.experimental.pallas import tpu_sc as plsc

VOCAB_SIZE = 100000
D_MODEL = 128
BATCH = 4096
SEQ_LEN = 50
SCALE = math.sqrt(D_MODEL)

NC = 2
NS = 16
NW = NC * NS

CHUNK = BATCH // NW
NBUF = 5


def _body(x_hbm, w_hbm, out_hbm, idx_v, rows_v, gsem, ssem):
    wid = lax.axis_index("s") * NC + lax.axis_index("c")
    pltpu.sync_copy(x_hbm.at[:, wid], idx_v)

    def gather(l, buf, wait):
        src = w_hbm.at[idx_v.at[l]]
        dst = rows_v.at[buf]
        if wait:
            pltpu.make_async_copy(src, dst, gsem).wait()
        else:
            pltpu.async_copy(src, dst, gsem)

    def scatter(l, buf, wait):
        src = rows_v.at[buf]
        dst = out_hbm.at[pl.ds(l * BATCH + wid * CHUNK, CHUNK)]
        if wait:
            pltpu.make_async_copy(src, dst, ssem).wait()
        else:
            pltpu.async_copy(src, dst, ssem)

    for b in range(NBUF - 1):
        gather(b, b, wait=False)

    @pl.loop(0, SEQ_LEN, step=NBUF)
    def outer(l0):
        for k in range(NBUF):
            l = l0 + k
            prev = (k - 1) % NBUF
            gather(l, k, wait=True)

            @pl.when(l > 0)
            def _():
                scatter(l - 1, prev, wait=True)

            @pl.when(l + NBUF - 1 < SEQ_LEN)
            def _():
                gather(l + NBUF - 1, prev, wait=False)

            @plsc.parallel_loop(0, CHUNK, unroll=8)
            def scale_row(r):
                for j in range(D_MODEL // 16):
                    rows_v[k, r, pl.ds(j * 16, 16)] = (
                        rows_v[k, r, pl.ds(j * 16, 16)] * SCALE
                    )

            scatter(l, k, wait=False)

    scatter(SEQ_LEN - 1, (SEQ_LEN - 1) % NBUF, wait=True)


@jax.jit
def kernel(x, weight):
    xt = x.T.reshape(SEQ_LEN, NW, CHUNK)
    mesh = plsc.VectorSubcoreMesh(
        core_axis_name="c", subcore_axis_name="s", num_cores=NC, num_subcores=NS
    )
    out = pl.kernel(
        _body,
        out_type=jax.ShapeDtypeStruct((SEQ_LEN * BATCH, D_MODEL), jnp.float32),
        mesh=mesh,
        scratch_types=[
            pltpu.VMEM((SEQ_LEN, CHUNK), jnp.int32),
            pltpu.VMEM((NBUF, CHUNK, D_MODEL), jnp.float32),
            pltpu.SemaphoreType.DMA,
            pltpu.SemaphoreType.DMA,
        ],
    )(xt, weight)
    return out.reshape(SEQ_LEN, BATCH, D_MODEL).transpose(1, 0, 2)
